# Initial kernel scaffold; baseline (speedup 1.0000x reference)
import jax
import jax.numpy as jnp
from jax import lax
from jax.experimental import pallas as pl
from jax.experimental.pallas import tpu as pltpu
from jax.experimental.pallas import tpu_sc as plsc

N = 10000
C = 128
E = 320000
NS = 16
K = 80
EPT = E // NS
CHUNKS = EPT // K
RPT = N // NS
CW = 16


def _make_agg(with_counts):
  out_type = [
      jax.ShapeDtypeStruct((N, C), jnp.float32),
      jax.ShapeDtypeStruct((N, C), jnp.float32),
  ]
  if with_counts:
    out_type += [
        jax.ShapeDtypeStruct((N, CW), jnp.float32),
        jax.ShapeDtypeStruct((N, CW), jnp.float32),
    ]
  scratch = [
      pltpu.VMEM((CHUNKS, K), jnp.int32),
      pltpu.VMEM((CHUNKS, K), jnp.int32),
      pltpu.VMEM((K, C), jnp.float32),
      pltpu.VMEM_SHARED((N, C), jnp.float32),
      pltpu.SemaphoreType.DMA,
  ]
  if with_counts:
    scratch += [
        pltpu.VMEM((K, CW), jnp.float32),
        pltpu.VMEM_SHARED((N, CW), jnp.float32),
    ]

  def body(xu, xi, src_ui, dst_ui, src_iu, dst_iu, z128, z16, *refs):
    if with_counts:
      sum_i, sum_u, cnt_i, cnt_u = refs[:4]
      src_v, dst_v, rows_v, acc, sem, ones_v, cacc = refs[4:]
    else:
      sum_i, sum_u = refs[:2]
      src_v, dst_v, rows_v, acc, sem = refs[2:]
      cnt_i = cnt_u = cacc = ones_v = None
    c = lax.axis_index("c")
    s = lax.axis_index("s")
    r0 = s * RPT

    def run(x_hbm, src_hbm, dst_hbm, sum_out, cnt_out):
      pltpu.sync_copy(z128.at[pl.ds(r0, RPT)], acc.at[pl.ds(r0, RPT)])
      if with_counts:
        pltpu.sync_copy(z16.at[pl.ds(r0, RPT)], cacc.at[pl.ds(r0, RPT)])

        def fill(i, carry):
          ones_v[i] = jnp.ones((CW,), jnp.float32)
          return carry
        lax.fori_loop(0, K, fill, 0)
      pltpu.sync_copy(src_hbm.at[pl.ds(s * CHUNKS, CHUNKS)], src_v)
      pltpu.sync_copy(dst_hbm.at[pl.ds(s * CHUNKS, CHUNKS)], dst_v)
      plsc.subcore_barrier()

      def chunk(g, carry):
        pltpu.async_copy(x_hbm.at[src_v.at[g]], rows_v, sem).wait()
        pltpu.sync_copy(rows_v, acc.at[dst_v.at[g]], add=True)
        if with_counts:
          pltpu.sync_copy(ones_v, cacc.at[dst_v.at[g]], add=True)
        return carry
      lax.fori_loop(0, CHUNKS, chunk, 0)
      plsc.subcore_barrier()
      pltpu.sync_copy(acc.at[pl.ds(r0, RPT)], sum_out.at[pl.ds(r0, RPT)])
      if with_counts:
        pltpu.sync_copy(cacc.at[pl.ds(r0, RPT)], cnt_out.at[pl.ds(r0, RPT)])

    @pl.when(c == 0)
    def _():
      run(xu, src_ui, dst_ui, sum_i, cnt_i)

    @pl.when(c == 1)
    def _():
      run(xi, src_iu, dst_iu, sum_u, cnt_u)

  mesh = plsc.VectorSubcoreMesh(core_axis_name="c", subcore_axis_name="s")
  return pl.kernel(body, out_type=out_type, mesh=mesh,
                   scratch_types=scratch,
                   name="sc_segment_sum" + ("_cnt" if with_counts else ""))


_BLK = 1000


def _ln_relu(x, scale, bias, eps=1e-5):
  mu = jnp.mean(x, axis=-1, keepdims=True)
  var = jnp.mean((x - mu) ** 2, axis=-1, keepdims=True)
  return jnp.maximum((x - mu) / jnp.sqrt(var + eps) * scale + bias, 0.0)


def _mean(sum_ref, cnt_ref):
  cnt = cnt_ref[...][:, :1]
  return sum_ref[...] / jnp.maximum(cnt, 1.0)


def _dense1_body(xu, xi, su, si, cu, ci, wsu, wsi, wru2i, wri2u, btu, bti,
                 lnsu, lnbu, lnsi, lnbi, xu_out, xi_out):
  aggu = _mean(su, cu)
  aggi = _mean(si, ci)
  nu = (jnp.dot(xu[...], wsu[...], preferred_element_type=jnp.float32)
        + jnp.dot(aggu, wri2u[...], preferred_element_type=jnp.float32)
        + btu[...])
  ni = (jnp.dot(xi[...], wsi[...], preferred_element_type=jnp.float32)
        + jnp.dot(aggi, wru2i[...], preferred_element_type=jnp.float32)
        + bti[...])
  xu_out[...] = _ln_relu(nu, lnsu[...], lnbu[...])
  xi_out[...] = _ln_relu(ni, lnsi[...], lnbi[...])


def _dense2_body(xu, xi, su, si, cu, ci, wsu, wsi, wru2i, wri2u, btu, bti,
                 lnsu, lnbu, lnsi, lnbi, wlua, wlub, wlia, wlib, blu, bli,
                 l2su, l2bu, l2si, l2bi, ou_out, oi_out):
  aggu = _mean(su, cu)
  aggi = _mean(si, ci)
  nu = (jnp.dot(xu[...], wsu[...], preferred_element_type=jnp.float32)
        + jnp.dot(aggu, wri2u[...], preferred_element_type=jnp.float32)
        + btu[...])
  ni = (jnp.dot(xi[...], wsi[...], preferred_element_type=jnp.float32)
        + jnp.dot(aggi, wru2i[...], preferred_element_type=jnp.float32)
        + bti[...])
  xu2 = _ln_relu(nu, lnsu[...], lnbu[...])
  xi2 = _ln_relu(ni, lnsi[...], lnbi[...])
  ju = (jnp.dot(xu[...], wlua[...], preferred_element_type=jnp.float32)
        + jnp.dot(xu2, wlub[...], preferred_element_type=jnp.float32)
        + blu[...])
  ji = (jnp.dot(xi[...], wlia[...], preferred_element_type=jnp.float32)
        + jnp.dot(xi2, wlib[...], preferred_element_type=jnp.float32)
        + bli[...])
  ou_out[...] = _ln_relu(ju, l2su[...], l2bu[...])
  oi_out[...] = _ln_relu(ji, l2si[...], l2bi[...])


def _row_spec():
  return pl.BlockSpec((_BLK, C), lambda i: (i, 0))


def _cnt_spec():
  return pl.BlockSpec((_BLK, CW), lambda i: (i, 0))


def _full_spec(shape):
  return pl.BlockSpec(shape, lambda i, _shape=shape: tuple(0 for _ in _shape))


def _dense1(*args):
  in_specs = ([_row_spec()] * 4 + [_cnt_spec()] * 2
              + [_full_spec((C, C))] * 4 + [_full_spec((1, C))] * 6)
  return pl.pallas_call(
      _dense1_body,
      grid=(N // _BLK,),
      in_specs=in_specs,
      out_specs=[_row_spec(), _row_spec()],
      out_shape=[jax.ShapeDtypeStruct((N, C), jnp.float32)] * 2,
  )(*args)


def _dense2(*args):
  in_specs = ([_row_spec()] * 4 + [_cnt_spec()] * 2
              + [_full_spec((C, C))] * 4 + [_full_spec((1, C))] * 6
              + [_full_spec((C, C))] * 4 + [_full_spec((1, C))] * 6)
  return pl.pallas_call(
      _dense2_body,
      grid=(N // _BLK,),
      in_specs=in_specs,
      out_specs=[_row_spec(), _row_spec()],
      out_shape=[jax.ShapeDtypeStruct((N, C), jnp.float32)] * 2,
  )(*args)


def kernel(x_user, x_item, edge_user_item, edge_item_user, W_self_user,
           b_self_user, W_self_item, b_self_item, W_rel_u2i, b_rel_u2i,
           W_rel_i2u, b_rel_i2u, ln_scale_user, ln_bias_user, ln_scale_item,
           ln_bias_item, W_lin_user, b_lin_user, ln2_scale_user,
           ln2_bias_user, W_lin_item, b_lin_item, ln2_scale_item,
           ln2_bias_item):
  src_ui = edge_user_item[0].reshape(E // K, K)
  dst_ui = edge_user_item[1].reshape(E // K, K)
  src_iu = edge_item_user[0].reshape(E // K, K)
  dst_iu = edge_item_user[1].reshape(E // K, K)
  z128 = jnp.zeros((N, C), jnp.float32)
  z16 = jnp.zeros((N, CW), jnp.float32)

  agg_cnt = _make_agg(True)
  agg_nocnt = _make_agg(False)

  sum_i0, sum_u0, cnt_i, cnt_u = agg_cnt(
      x_user, x_item, src_ui, dst_ui, src_iu, dst_iu, z128, z16)

  def row(v):
    return v.reshape(1, C)

  xu1, xi1 = _dense1(
      x_user, x_item, sum_u0, sum_i0, cnt_u, cnt_i,
      W_self_user[0], W_self_item[0], W_rel_u2i[0], W_rel_i2u[0],
      row(b_self_user[0] + b_rel_i2u[0]), row(b_self_item[0] + b_rel_u2i[0]),
      row(ln_scale_user[0]), row(ln_bias_user[0]),
      row(ln_scale_item[0]), row(ln_bias_item[0]))

  sum_i1, sum_u1 = agg_nocnt(
      xu1, xi1, src_ui, dst_ui, src_iu, dst_iu, z128, z16)

  ou, oi = _dense2(
      xu1, xi1, sum_u1, sum_i1, cnt_u, cnt_i,
      W_self_user[1], W_self_item[1], W_rel_u2i[1], W_rel_i2u[1],
      row(b_self_user[1] + b_rel_i2u[1]), row(b_self_item[1] + b_rel_u2i[1]),
      row(ln_scale_user[1]), row(ln_bias_user[1]),
      row(ln_scale_item[1]), row(ln_bias_item[1]),
      W_lin_user[:C], W_lin_user[C:], W_lin_item[:C], W_lin_item[C:],
      row(b_lin_user), row(b_lin_item),
      row(ln2_scale_user), row(ln2_bias_user),
      row(ln2_scale_item), row(ln2_bias_item))

  return jnp.concatenate([ou, oi], axis=0)


# speedup vs baseline: 2.8306x; 2.8306x over previous
import jax
import jax.numpy as jnp
from jax import lax
from jax.experimental import pallas as pl
from jax.experimental.pallas import tpu as pltpu
from jax.experimental.pallas import tpu_sc as plsc

N = 10000
NP = 10240
C = 128
E = 320000
NS = 16
K = 128
EP = 327680
ROWS = EP // K
CHUNKS = ROWS // NS
SB = 32
BLOCKS = CHUNKS // SB
RPT = NP // NS
CW = 16


def _make_agg():
  out_type = [
      jax.ShapeDtypeStruct((NP, C), jnp.float32),
      jax.ShapeDtypeStruct((NP, C), jnp.float32),
  ]
  scratch = [
      pltpu.VMEM((SB, K), jnp.int32),
      pltpu.VMEM((SB, K), jnp.int32),
      pltpu.VMEM((K, C), jnp.float32),
      pltpu.VMEM_SHARED((NP, C), jnp.float32),
      pltpu.SemaphoreType.DMA,
  ]

  def body(xu, xi, src_ui, dst_ui, src_iu, dst_iu, z128,
           sum_i, sum_u, src_blk, dst_blk, rows_v, acc, sem):
    c = lax.axis_index("c")
    s = lax.axis_index("s")
    r0 = s * RPT

    def run(x_hbm, src_hbm, dst_hbm, sum_out):
      pltpu.sync_copy(z128.at[pl.ds(r0, RPT)], acc.at[pl.ds(r0, RPT)])
      plsc.subcore_barrier()

      def block(b, carry):
        i0 = s * CHUNKS + b * SB
        pltpu.sync_copy(src_hbm.at[pl.ds(i0, SB)], src_blk)
        pltpu.sync_copy(dst_hbm.at[pl.ds(i0, SB)], dst_blk)

        def chunk(g, carry2):
          pltpu.async_copy(x_hbm.at[src_blk.at[g]], rows_v, sem).wait()
          pltpu.sync_copy(rows_v, acc.at[dst_blk.at[g]], add=True)
          return carry2
        lax.fori_loop(0, SB, chunk, 0)
        return carry
      lax.fori_loop(0, BLOCKS, block, 0)
      plsc.subcore_barrier()
      pltpu.sync_copy(acc.at[pl.ds(r0, RPT)], sum_out.at[pl.ds(r0, RPT)])

    @pl.when(c == 0)
    def _():
      run(xu, src_ui, dst_ui, sum_i)

    @pl.when(c == 1)
    def _():
      run(xi, src_iu, dst_iu, sum_u)

  mesh = plsc.VectorSubcoreMesh(core_axis_name="c", subcore_axis_name="s")
  return pl.kernel(body, out_type=out_type, mesh=mesh,
                   scratch_types=scratch, name="sc_segment_sum")


def _make_counts():
  out_type = [jax.ShapeDtypeStruct((NP, C), jnp.float32)] * 2
  scratch = [
      pltpu.VMEM((SB, K), jnp.int32),
      pltpu.VMEM((K, C), jnp.float32),
      pltpu.VMEM_SHARED((NP, C), jnp.float32),
  ]

  def body(dst_ui, dst_iu, z128, o128, cnt_i, cnt_u, dst_blk, ones_v, cacc):
    c = lax.axis_index("c")
    s = lax.axis_index("s")
    r0 = s * RPT

    def run(dst_hbm, cnt_out):
      pltpu.sync_copy(z128.at[pl.ds(r0, RPT)], cacc.at[pl.ds(r0, RPT)])
      pltpu.sync_copy(o128, ones_v)
      plsc.subcore_barrier()

      def block(b, carry):
        pltpu.sync_copy(dst_hbm.at[pl.ds(s * CHUNKS + b * SB, SB)], dst_blk)

        def chunk(g, carry2):
          pltpu.sync_copy(ones_v, cacc.at[dst_blk.at[g]], add=True)
          return carry2
        lax.fori_loop(0, SB, chunk, 0)
        return carry
      lax.fori_loop(0, BLOCKS, block, 0)
      plsc.subcore_barrier()
      pltpu.sync_copy(cacc.at[pl.ds(r0, RPT)], cnt_out.at[pl.ds(r0, RPT)])

    @pl.when(c == 0)
    def _():
      run(dst_ui, cnt_i)

    @pl.when(c == 1)
    def _():
      run(dst_iu, cnt_u)

  mesh = plsc.VectorSubcoreMesh(core_axis_name="c", subcore_axis_name="s")
  return pl.kernel(body, out_type=out_type, mesh=mesh,
                   scratch_types=scratch, name="sc_segment_count")


_BLK = 1000


def _ln_relu(x, scale, bias, eps=1e-5):
  mu = jnp.mean(x, axis=-1, keepdims=True)
  var = jnp.mean((x - mu) ** 2, axis=-1, keepdims=True)
  return jnp.maximum((x - mu) / jnp.sqrt(var + eps) * scale + bias, 0.0)


def _mean(sum_ref, cnt_ref):
  cnt = cnt_ref[...][:, :1]
  return sum_ref[...] * (1.0 / jnp.maximum(cnt, 1.0))


def _node_update(x, agg, ws, wr, bt, lns, lnb):
  n = (jnp.dot(x, ws[...], preferred_element_type=jnp.float32)
       + jnp.dot(agg, wr[...], preferred_element_type=jnp.float32)
       + bt[...])
  return _ln_relu(n, lns[...], lnb[...])


def _dense1_body(xu, xi, su, si, cu, ci, wsu, wsi, wru2i, wri2u,
                 btu, bti, lnsu, lnbu, lnsi, lnbi, xu_out, xi_out):
  xu_out[...] = _node_update(xu[...], _mean(su, cu), wsu, wri2u, btu,
                             lnsu, lnbu)
  xi_out[...] = _node_update(xi[...], _mean(si, ci), wsi, wru2i, bti,
                             lnsi, lnbi)


def _dense2_body(xu, xi, su, si, cu, ci, wsu, wsi, wru2i, wri2u, btu, bti,
                 lnsu, lnbu, lnsi, lnbi, wlua, wlub, wlia, wlib, blu, bli,
                 l2su, l2bu, l2si, l2bi, ou_out, oi_out):
  xu2 = _node_update(xu[...], _mean(su, cu), wsu, wri2u, btu, lnsu, lnbu)
  xi2 = _node_update(xi[...], _mean(si, ci), wsi, wru2i, bti, lnsi, lnbi)
  ju = (jnp.dot(xu[...], wlua[...], preferred_element_type=jnp.float32)
        + jnp.dot(xu2, wlub[...], preferred_element_type=jnp.float32)
        + blu[...])
  ji = (jnp.dot(xi[...], wlia[...], preferred_element_type=jnp.float32)
        + jnp.dot(xi2, wlib[...], preferred_element_type=jnp.float32)
        + bli[...])
  ou_out[...] = _ln_relu(ju, l2su[...], l2bu[...])
  oi_out[...] = _ln_relu(ji, l2si[...], l2bi[...])


def _row_spec():
  return pl.BlockSpec((_BLK, C), lambda i: (i, 0))


def _cnt_spec():
  return _row_spec()


def _full_spec(shape):
  return pl.BlockSpec(shape, lambda i, _shape=shape: tuple(0 for _ in _shape))


def _dense1(*args):
  in_specs = ([_row_spec()] * 4 + [_cnt_spec()] * 2
              + [_full_spec((C, C))] * 4 + [_full_spec((1, C))] * 6)
  return pl.pallas_call(
      _dense1_body,
      grid=(N // _BLK,),
      in_specs=in_specs,
      out_specs=[_row_spec()] * 2,
      out_shape=[jax.ShapeDtypeStruct((N, C), jnp.float32)] * 2,
  )(*args)


def _dense2(*args):
  in_specs = ([_row_spec()] * 4 + [_cnt_spec()] * 2
              + [_full_spec((C, C))] * 4 + [_full_spec((1, C))] * 6
              + [_full_spec((C, C))] * 4 + [_full_spec((1, C))] * 6)
  return pl.pallas_call(
      _dense2_body,
      grid=(N // _BLK,),
      in_specs=in_specs,
      out_specs=[_row_spec()] * 2,
      out_shape=[jax.ShapeDtypeStruct((N, C), jnp.float32)] * 2,
  )(*args)


def _pad_edges(e):
  npad = EP - E
  src = jnp.concatenate([e[0], jnp.zeros((npad,), jnp.int32)])
  dst = jnp.concatenate([e[1], jnp.full((npad,), N, jnp.int32)])
  return src.reshape(ROWS, K), dst.reshape(ROWS, K)


def kernel(x_user, x_item, edge_user_item, edge_item_user, W_self_user,
           b_self_user, W_self_item, b_self_item, W_rel_u2i, b_rel_u2i,
           W_rel_i2u, b_rel_i2u, ln_scale_user, ln_bias_user, ln_scale_item,
           ln_bias_item, W_lin_user, b_lin_user, ln2_scale_user,
           ln2_bias_user, W_lin_item, b_lin_item, ln2_scale_item,
           ln2_bias_item):
  src_ui, dst_ui = _pad_edges(edge_user_item)
  src_iu, dst_iu = _pad_edges(edge_item_user)
  z128 = jnp.zeros((NP, C), jnp.float32)
  o128 = jnp.ones((K, C), jnp.float32)

  agg = _make_agg()
  counts = _make_counts()

  cnt_i, cnt_u = counts(dst_ui, dst_iu, z128, o128)
  sum_i0, sum_u0 = agg(x_user, x_item, src_ui, dst_ui, src_iu, dst_iu, z128)

  def row(v):
    return v.reshape(1, C)

  xu1, xi1 = _dense1(
      x_user, x_item, sum_u0, sum_i0, cnt_u, cnt_i,
      W_self_user[0], W_self_item[0], W_rel_u2i[0], W_rel_i2u[0],
      row(b_self_user[0] + b_rel_i2u[0]), row(b_self_item[0] + b_rel_u2i[0]),
      row(ln_scale_user[0]), row(ln_bias_user[0]),
      row(ln_scale_item[0]), row(ln_bias_item[0]))

  sum_i1, sum_u1 = agg(xu1, xi1, src_ui, dst_ui, src_iu, dst_iu, z128)

  ou, oi = _dense2(
      xu1, xi1, sum_u1, sum_i1, cnt_u, cnt_i,
      W_self_user[1], W_self_item[1], W_rel_u2i[1], W_rel_i2u[1],
      row(b_self_user[1] + b_rel_i2u[1]), row(b_self_item[1] + b_rel_u2i[1]),
      row(ln_scale_user[1]), row(ln_bias_user[1]),
      row(ln_scale_item[1]), row(ln_bias_item[1]),
      W_lin_user[:C], W_lin_user[C:], W_lin_item[:C], W_lin_item[C:],
      row(b_lin_user), row(b_lin_item),
      row(ln2_scale_user), row(ln2_bias_user),
      row(ln2_scale_item), row(ln2_bias_item))

  return jnp.concatenate([ou, oi], axis=0)
